# baseline (device time: 27710 ns/iter reference)
import os

import jax
import jax.numpy as jnp
from jax import lax
from jax.experimental import pallas as pl
from jax.experimental.pallas import tpu as pltpu

_SKIP_COMM = os.environ.get("SKIP_COMM") == "1"

N_DEV = 16
STAGE_MASKS = (1, 3, 4, 8)
N_STAGES = len(STAGE_MASKS)

B, Sq, Hq, Hkv, Dh = 2, 128, 8, 2, 64
D = Hq * Dh
GROUP = Hq // Hkv
GSQ = GROUP * Sq
SCALE = 0.125


def kernel(x, Wq, Wo, K_ext, V_ext):
    def body(x_ref, wq_ref, wo_ref, k_ref, v_ref, out_ref,
             acc_ref, stats_ref, rbuf_ref, rstats_ref,
             kbuf_ref, vbuf_ref, qg_ref, obuf_ref,
             send_o, recv_o, send_s, recv_s):
        my = lax.axis_index("i")

        if not _SKIP_COMM:
            barrier_sem = pltpu.get_barrier_semaphore()
            for mask in STAGE_MASKS:
                pl.semaphore_signal(
                    barrier_sem, inc=1,
                    device_id=(my ^ mask,),
                    device_id_type=pl.DeviceIdType.MESH,
                )

        def partial(b):
            for g in range(Hkv):
                kbuf_ref[b, g] = k_ref[b, :, g, :]
                vbuf_ref[b, g] = v_ref[b, :, g, :]
            q_all = lax.dot_general(
                x_ref[b], wq_ref[...], (((1,), (0,)), ((), ())),
            )
            for h in range(Hq):
                g, hh = divmod(h, GROUP)
                qg_ref[b, g, hh * Sq:(hh + 1) * Sq, :] = (
                    q_all[:, h * Dh:(h + 1) * Dh]
                )
            st = lax.dot_general(
                kbuf_ref[b], qg_ref[b], (((2,), (2,)), ((0,), (0,))),
            ) * SCALE
            m = jnp.max(st, axis=1, keepdims=True)
            p = jnp.exp(st - m)
            l = jnp.sum(p, axis=1, keepdims=True)
            ot = lax.dot_general(
                vbuf_ref[b], p, (((1,), (1,)), ((0,), (0,))),
            )
            acc_ref[0, b] = ot.astype(jnp.bfloat16)
            stats_ref[b, 0] = m
            stats_ref[b, 1] = l

        def send(s, t):
            partner = my ^ STAGE_MASKS[s]
            st_rdma = pltpu.make_async_remote_copy(
                src_ref=stats_ref.at[t],
                dst_ref=rstats_ref.at[s * B + t],
                send_sem=send_s.at[s, t],
                recv_sem=recv_s.at[s, t],
                device_id=(partner,),
                device_id_type=pl.DeviceIdType.MESH,
            )
            out_rdma = pltpu.make_async_remote_copy(
                src_ref=acc_ref.at[s % 2, t],
                dst_ref=rbuf_ref.at[s * B + t],
                send_sem=send_o.at[s, t],
                recv_sem=recv_o.at[s, t],
                device_id=(partner,),
                device_id_type=pl.DeviceIdType.MESH,
            )
            st_rdma.start()
            out_rdma.start()
            return st_rdma, out_rdma

        def combine(s, t, rdmas, prev):
            st_rdma, out_rdma = rdmas
            p = s % 2
            st_rdma.wait()
            m_a = stats_ref[t, 0]
            l_a = stats_ref[t, 1]
            m_b = rstats_ref[s * B + t, 0]
            l_b = rstats_ref[s * B + t, 1]
            m_n = jnp.maximum(m_a, m_b)
            a_a = jnp.exp(m_a - m_n)
            a_b = jnp.exp(m_b - m_n)
            stats_ref[t, 0] = m_n
            stats_ref[t, 1] = l_a * a_a + l_b * a_b
            if prev is not None:
                prev[1].wait_send()
            out_rdma.wait_recv()
            acc_ref[1 - p, t] = (
                acc_ref[p, t].astype(jnp.float32) * a_a
                + rbuf_ref[s * B + t].astype(jnp.float32) * a_b
            ).astype(jnp.bfloat16)

        def project(b):
            slot = N_STAGES % 2
            for h in range(Hq):
                g, hh = divmod(h, GROUP)
                inv_l = 1.0 / stats_ref[b, 1, g, :, hh * Sq:(hh + 1) * Sq]
                obuf_ref[h * Dh:(h + 1) * Dh, :] = (
                    acc_ref[slot, b, g, :, hh * Sq:(hh + 1) * Sq].astype(
                        jnp.float32
                    ) * inv_l
                )
            out_ref[b] = lax.dot_general(
                obuf_ref[...], wo_ref[...], (((0,), (0,)), ((), ())),
            )

        if _SKIP_COMM:
            partial(0)
            partial(1)
            project(0)
            project(1)
            return

        partial(0)
        pl.semaphore_wait(barrier_sem, N_STAGES)
        r = [[None, None] for _ in range(N_STAGES)]
        r[0][0] = send(0, 0)
        partial(1)
        r[0][1] = send(0, 1)
        for s in range(N_STAGES - 1):
            combine(s, 0, r[s][0], r[s - 1][0] if s else None)
            r[s + 1][0] = send(s + 1, 0)
            combine(s, 1, r[s][1], r[s - 1][1] if s else None)
            r[s + 1][1] = send(s + 1, 1)
        last = N_STAGES - 1
        combine(last, 0, r[last][0], r[last - 1][0])
        project(0)
        combine(last, 1, r[last][1], r[last - 1][1])
        project(1)
        r[last][0][1].wait_send()
        r[last][1][1].wait_send()

    return pl.pallas_call(
        body,
        out_shape=jax.ShapeDtypeStruct((B, Sq, D), jnp.float32),
        in_specs=[pl.BlockSpec(memory_space=pltpu.VMEM)] * 5,
        out_specs=pl.BlockSpec(memory_space=pltpu.VMEM),
        scratch_shapes=[
            pltpu.VMEM((2, B, Hkv, Dh, GSQ), jnp.bfloat16),
            pltpu.VMEM((B, 2, Hkv, 1, GSQ), jnp.float32),
            pltpu.VMEM((N_STAGES * B, Hkv, Dh, GSQ), jnp.bfloat16),
            pltpu.VMEM((N_STAGES * B, 2, Hkv, 1, GSQ), jnp.float32),
            pltpu.VMEM((B, Hkv, Sq, Dh), jnp.float32),
            pltpu.VMEM((B, Hkv, Sq, Dh), jnp.float32),
            pltpu.VMEM((B, Hkv, GSQ, Dh), jnp.float32),
            pltpu.VMEM((D, Sq), jnp.float32),
            pltpu.SemaphoreType.DMA((N_STAGES, B)),
            pltpu.SemaphoreType.DMA((N_STAGES, B)),
            pltpu.SemaphoreType.DMA((N_STAGES, B)),
            pltpu.SemaphoreType.DMA((N_STAGES, B)),
        ],
        compiler_params=(
            None if _SKIP_COMM else pltpu.CompilerParams(collective_id=0)
        ),
    )(x, Wq, Wo, K_ext, V_ext)


# device time: 25952 ns/iter; 1.0677x vs baseline; 1.0677x over previous
import os

import jax
import jax.numpy as jnp
from jax import lax
from jax.experimental import pallas as pl
from jax.experimental.pallas import tpu as pltpu

_SKIP_COMM = os.environ.get("SKIP_COMM") == "1"

N_DEV = 16
STAGE_MASKS = (8, 3, 4, 1)
N_STAGES = len(STAGE_MASKS)

B, Sq, Hq, Hkv, Dh = 2, 128, 8, 2, 64
D = Hq * Dh
GROUP = Hq // Hkv
GSQ = GROUP * Sq
SCALE = 0.125


def kernel(x, Wq, Wo, K_ext, V_ext):
    def body(x_ref, wq_ref, wo_ref, k_ref, v_ref, out_ref,
             acc_ref, stats_ref, rbuf_ref, rstats_ref,
             kbuf_ref, vbuf_ref, qg_ref, obuf_ref,
             send_o, recv_o, send_s, recv_s):
        my = lax.axis_index("i")

        if not _SKIP_COMM:
            barrier_sem = pltpu.get_barrier_semaphore()
            for mask in STAGE_MASKS:
                pl.semaphore_signal(
                    barrier_sem, inc=1,
                    device_id=(my ^ mask,),
                    device_id_type=pl.DeviceIdType.MESH,
                )

        def partial(b):
            for g in range(Hkv):
                kbuf_ref[b, g] = k_ref[b, :, g, :]
                vbuf_ref[b, g] = v_ref[b, :, g, :]
            q_all = lax.dot_general(
                x_ref[b], wq_ref[...], (((1,), (0,)), ((), ())),
            ) * SCALE
            for h in range(Hq):
                g, hh = divmod(h, GROUP)
                qg_ref[b, g, hh * Sq:(hh + 1) * Sq, :] = (
                    q_all[:, h * Dh:(h + 1) * Dh]
                )
            st = lax.dot_general(
                kbuf_ref[b], qg_ref[b], (((2,), (2,)), ((0,), (0,))),
            )
            m = jnp.max(st, axis=1, keepdims=True)
            p = jnp.exp(st - m)
            l = jnp.sum(p, axis=1, keepdims=True)
            ot = lax.dot_general(
                vbuf_ref[b], p, (((1,), (1,)), ((0,), (0,))),
            )
            acc_ref[0, b] = ot.astype(jnp.bfloat16)
            stats_ref[b, 0] = m
            stats_ref[b, 1] = l

        def send(s, t):
            partner = my ^ STAGE_MASKS[s]
            st_rdma = pltpu.make_async_remote_copy(
                src_ref=stats_ref.at[t],
                dst_ref=rstats_ref.at[s * B + t],
                send_sem=send_s.at[s, t],
                recv_sem=recv_s.at[s, t],
                device_id=(partner,),
                device_id_type=pl.DeviceIdType.MESH,
            )
            out_rdma = pltpu.make_async_remote_copy(
                src_ref=acc_ref.at[s % 2, t],
                dst_ref=rbuf_ref.at[s * B + t],
                send_sem=send_o.at[s, t],
                recv_sem=recv_o.at[s, t],
                device_id=(partner,),
                device_id_type=pl.DeviceIdType.MESH,
            )
            st_rdma.start()
            out_rdma.start()
            return st_rdma, out_rdma

        def combine(s, t, rdmas, prev):
            st_rdma, out_rdma = rdmas
            p = s % 2
            st_rdma.wait()
            m_a = stats_ref[t, 0]
            l_a = stats_ref[t, 1]
            m_b = rstats_ref[s * B + t, 0]
            l_b = rstats_ref[s * B + t, 1]
            m_n = jnp.maximum(m_a, m_b)
            a_a = jnp.exp(m_a - m_n)
            a_b = jnp.exp(m_b - m_n)
            stats_ref[t, 0] = m_n
            stats_ref[t, 1] = l_a * a_a + l_b * a_b
            if prev is not None:
                prev[1].wait_send()
            out_rdma.wait_recv()
            acc_ref[1 - p, t] = (
                acc_ref[p, t].astype(jnp.float32) * a_a
                + rbuf_ref[s * B + t].astype(jnp.float32) * a_b
            ).astype(jnp.bfloat16)

        def project(b):
            slot = N_STAGES % 2
            for h in range(Hq):
                g, hh = divmod(h, GROUP)
                inv_l = 1.0 / stats_ref[b, 1, g, :, hh * Sq:(hh + 1) * Sq]
                obuf_ref[h * Dh:(h + 1) * Dh, :] = (
                    acc_ref[slot, b, g, :, hh * Sq:(hh + 1) * Sq].astype(
                        jnp.float32
                    ) * inv_l
                )
            out_ref[b] = lax.dot_general(
                obuf_ref[...], wo_ref[...], (((0,), (0,)), ((), ())),
            )

        if _SKIP_COMM:
            partial(0)
            partial(1)
            project(0)
            project(1)
            return

        partial(0)
        pl.semaphore_wait(barrier_sem, N_STAGES)
        r = [[None, None] for _ in range(N_STAGES)]
        r[0][0] = send(0, 0)
        partial(1)
        r[0][1] = send(0, 1)
        for s in range(N_STAGES - 1):
            combine(s, 0, r[s][0], r[s - 1][0] if s else None)
            r[s + 1][0] = send(s + 1, 0)
            combine(s, 1, r[s][1], r[s - 1][1] if s else None)
            r[s + 1][1] = send(s + 1, 1)
        last = N_STAGES - 1
        combine(last, 0, r[last][0], r[last - 1][0])
        project(0)
        combine(last, 1, r[last][1], r[last - 1][1])
        project(1)
        r[last][0][1].wait_send()
        r[last][1][1].wait_send()

    return pl.pallas_call(
        body,
        out_shape=jax.ShapeDtypeStruct((B, Sq, D), jnp.float32),
        in_specs=[pl.BlockSpec(memory_space=pltpu.VMEM)] * 5,
        out_specs=pl.BlockSpec(memory_space=pltpu.VMEM),
        scratch_shapes=[
            pltpu.VMEM((2, B, Hkv, Dh, GSQ), jnp.bfloat16),
            pltpu.VMEM((B, 2, Hkv, 1, GSQ), jnp.float32),
            pltpu.VMEM((N_STAGES * B, Hkv, Dh, GSQ), jnp.bfloat16),
            pltpu.VMEM((N_STAGES * B, 2, Hkv, 1, GSQ), jnp.float32),
            pltpu.VMEM((B, Hkv, Sq, Dh), jnp.float32),
            pltpu.VMEM((B, Hkv, Sq, Dh), jnp.float32),
            pltpu.VMEM((B, Hkv, GSQ, Dh), jnp.float32),
            pltpu.VMEM((D, Sq), jnp.float32),
            pltpu.SemaphoreType.DMA((N_STAGES, B)),
            pltpu.SemaphoreType.DMA((N_STAGES, B)),
            pltpu.SemaphoreType.DMA((N_STAGES, B)),
            pltpu.SemaphoreType.DMA((N_STAGES, B)),
        ],
        compiler_params=(
            None if _SKIP_COMM else pltpu.CompilerParams(collective_id=0)
        ),
    )(x, Wq, Wo, K_ext, V_ext)


# device time: 25331 ns/iter; 1.0939x vs baseline; 1.0245x over previous
import os

import jax
import jax.numpy as jnp
from jax import lax
from jax.experimental import pallas as pl
from jax.experimental.pallas import tpu as pltpu

_SKIP_COMM = os.environ.get("SKIP_COMM") == "1"

N_DEV = 16
STAGE_MASKS = (8, 3, 4, 1)
N_STAGES = len(STAGE_MASKS)

B, Sq, Hq, Hkv, Dh = 2, 128, 8, 2, 64
D = Hq * Dh
GROUP = Hq // Hkv
GSQ = GROUP * Sq
SCALE = 0.125
STREAMS = tuple((b, g) for b in range(B) for g in range(Hkv))


def kernel(x, Wq, Wo, K_ext, V_ext):
    def body(x_ref, wq_ref, wo_ref, k_ref, v_ref, out_ref,
             acc_ref, stats_ref, rbuf_ref, rstats_ref,
             kbuf_ref, vbuf_ref, qg_ref, obuf_ref,
             send_o, recv_o, send_s, recv_s):
        my = lax.axis_index("i")

        if not _SKIP_COMM:
            barrier_sem = pltpu.get_barrier_semaphore()
            for mask in STAGE_MASKS:
                pl.semaphore_signal(
                    barrier_sem, inc=1,
                    device_id=(my ^ mask,),
                    device_id_type=pl.DeviceIdType.MESH,
                )

        def partial(b):
            for g in range(Hkv):
                kbuf_ref[b, g] = k_ref[b, :, g, :]
                vbuf_ref[b, g] = v_ref[b, :, g, :]
            q_all = lax.dot_general(
                x_ref[b], wq_ref[...], (((1,), (0,)), ((), ())),
            ) * SCALE
            for h in range(Hq):
                g, hh = divmod(h, GROUP)
                qg_ref[b, g, hh * Sq:(hh + 1) * Sq, :] = (
                    q_all[:, h * Dh:(h + 1) * Dh]
                )
            st = lax.dot_general(
                kbuf_ref[b], qg_ref[b], (((2,), (2,)), ((0,), (0,))),
            )
            m = jnp.max(st, axis=1, keepdims=True)
            p = jnp.exp(st - m)
            l = jnp.sum(p, axis=1, keepdims=True)
            ot = lax.dot_general(
                vbuf_ref[b], p, (((1,), (1,)), ((0,), (0,))),
            )
            acc_ref[0, b] = ot.astype(jnp.bfloat16)
            stats_ref[b, :, 0] = m
            stats_ref[b, :, 1] = l

        def send(s, b, g):
            partner = my ^ STAGE_MASKS[s]
            st_rdma = pltpu.make_async_remote_copy(
                src_ref=stats_ref.at[b, g],
                dst_ref=rstats_ref.at[s * B + b, g],
                send_sem=send_s.at[s, b, g],
                recv_sem=recv_s.at[s, b, g],
                device_id=(partner,),
                device_id_type=pl.DeviceIdType.MESH,
            )
            out_rdma = pltpu.make_async_remote_copy(
                src_ref=acc_ref.at[s % 2, b, g],
                dst_ref=rbuf_ref.at[s * B + b, g],
                send_sem=send_o.at[s, b, g],
                recv_sem=recv_o.at[s, b, g],
                device_id=(partner,),
                device_id_type=pl.DeviceIdType.MESH,
            )
            st_rdma.start()
            out_rdma.start()
            return st_rdma, out_rdma

        def combine(s, b, g, rdmas, prev):
            st_rdma, out_rdma = rdmas
            p = s % 2
            st_rdma.wait()
            m_a = stats_ref[b, g, 0]
            l_a = stats_ref[b, g, 1]
            m_b = rstats_ref[s * B + b, g, 0]
            l_b = rstats_ref[s * B + b, g, 1]
            m_n = jnp.maximum(m_a, m_b)
            a_a = jnp.exp(m_a - m_n)
            a_b = jnp.exp(m_b - m_n)
            stats_ref[b, g, 0] = m_n
            stats_ref[b, g, 1] = l_a * a_a + l_b * a_b
            if prev is not None:
                prev[1].wait_send()
            out_rdma.wait_recv()
            acc_ref[1 - p, b, g] = (
                acc_ref[p, b, g].astype(jnp.float32) * a_a
                + rbuf_ref[s * B + b, g].astype(jnp.float32) * a_b
            ).astype(jnp.bfloat16)

        def project(b):
            slot = N_STAGES % 2
            for h in range(Hq):
                g, hh = divmod(h, GROUP)
                inv_l = 1.0 / stats_ref[b, g, 1, :, hh * Sq:(hh + 1) * Sq]
                obuf_ref[h * Dh:(h + 1) * Dh, :] = (
                    acc_ref[slot, b, g, :, hh * Sq:(hh + 1) * Sq].astype(
                        jnp.float32
                    ) * inv_l
                )
            out_ref[b] = lax.dot_general(
                obuf_ref[...], wo_ref[...], (((0,), (0,)), ((), ())),
            )

        if _SKIP_COMM:
            partial(0)
            partial(1)
            project(0)
            project(1)
            return

        partial(0)
        pl.semaphore_wait(barrier_sem, N_STAGES)
        r = [{} for _ in range(N_STAGES)]
        r[0][(0, 0)] = send(0, 0, 0)
        r[0][(0, 1)] = send(0, 0, 1)
        partial(1)
        r[0][(1, 0)] = send(0, 1, 0)
        r[0][(1, 1)] = send(0, 1, 1)
        for s in range(N_STAGES - 1):
            for bg in STREAMS:
                combine(s, *bg, r[s][bg], r[s - 1][bg] if s else None)
                r[s + 1][bg] = send(s + 1, *bg)
        last = N_STAGES - 1
        combine(last, 0, 0, r[last][(0, 0)], r[last - 1][(0, 0)])
        combine(last, 0, 1, r[last][(0, 1)], r[last - 1][(0, 1)])
        project(0)
        combine(last, 1, 0, r[last][(1, 0)], r[last - 1][(1, 0)])
        combine(last, 1, 1, r[last][(1, 1)], r[last - 1][(1, 1)])
        project(1)
        for bg in STREAMS:
            r[last][bg][1].wait_send()

    return pl.pallas_call(
        body,
        out_shape=jax.ShapeDtypeStruct((B, Sq, D), jnp.float32),
        in_specs=[pl.BlockSpec(memory_space=pltpu.VMEM)] * 5,
        out_specs=pl.BlockSpec(memory_space=pltpu.VMEM),
        scratch_shapes=[
            pltpu.VMEM((2, B, Hkv, Dh, GSQ), jnp.bfloat16),
            pltpu.VMEM((B, Hkv, 2, 1, GSQ), jnp.float32),
            pltpu.VMEM((N_STAGES * B, Hkv, Dh, GSQ), jnp.bfloat16),
            pltpu.VMEM((N_STAGES * B, Hkv, 2, 1, GSQ), jnp.float32),
            pltpu.VMEM((B, Hkv, Sq, Dh), jnp.float32),
            pltpu.VMEM((B, Hkv, Sq, Dh), jnp.float32),
            pltpu.VMEM((B, Hkv, GSQ, Dh), jnp.float32),
            pltpu.VMEM((D, Sq), jnp.float32),
            pltpu.SemaphoreType.DMA((N_STAGES, B, Hkv)),
            pltpu.SemaphoreType.DMA((N_STAGES, B, Hkv)),
            pltpu.SemaphoreType.DMA((N_STAGES, B, Hkv)),
            pltpu.SemaphoreType.DMA((N_STAGES, B, Hkv)),
        ],
        compiler_params=(
            None if _SKIP_COMM else pltpu.CompilerParams(collective_id=0)
        ),
    )(x, Wq, Wo, K_ext, V_ext)
